# baseline (speedup 1.0000x reference)
import jax
import jax.numpy as jnp
import numpy as np
from jax.experimental import pallas as pl
from jax.experimental.pallas import tpu as pltpu

_LTM_SIZE = 1000000
_EMB_DIM = 128
_BATCH = 32
_TEMPERATURE = 0.11 - float(np.log10(float(_LTM_SIZE))) * 0.01
_INV_T = 1.0 / _TEMPERATURE

_CHUNK = 8000
_NCHUNK = _LTM_SIZE // _CHUNK
_NSPLIT = 4
_SUB = _CHUNK // _NSPLIT


def _attn_kernel(q_ref, *refs):
    kv_refs = refs[:2 * _NSPLIT]
    o_ref, acc_ref, den_ref = refs[2 * _NSPLIT:]
    j = pl.program_id(0)

    q = q_ref[...]
    norm = jnp.sqrt(jnp.sum(q * q, axis=1, keepdims=True))
    qs = (q / jnp.maximum(norm, 1e-12)) * _INV_T

    pv = jnp.zeros((_BATCH, _EMB_DIM), jnp.float32)
    ps = jnp.zeros((_BATCH, 1), jnp.float32)
    for i in range(_NSPLIT):
        k_ref = kv_refs[i]
        v_ref = kv_refs[_NSPLIT + i]
        s = jax.lax.dot_general(
            qs, k_ref[...], (((1,), (1,)), ((), ())),
            preferred_element_type=jnp.float32,
        )
        p = jnp.exp(s - _INV_T)
        pv = pv + jax.lax.dot_general(
            p, v_ref[...], (((1,), (0,)), ((), ())),
            preferred_element_type=jnp.float32,
        )
        ps = ps + jnp.sum(p, axis=1, keepdims=True)
    psum = jnp.broadcast_to(ps, (_BATCH, _EMB_DIM))

    @pl.when(j == 0)
    def _init():
        acc_ref[...] = pv
        den_ref[...] = psum

    @pl.when(j != 0)
    def _accum():
        acc_ref[...] += pv
        den_ref[...] += psum

    @pl.when(j == _NCHUNK - 1)
    def _finish():
        o_ref[...] = acc_ref[...] / den_ref[...]


def _sub_spec(i):
    return pl.BlockSpec((_SUB, _EMB_DIM), lambda j, i=i: (_NSPLIT * j + i, 0))


def kernel(encoded_state, keys, values):
    sub_specs = [_sub_spec(i) for i in range(_NSPLIT)]
    return pl.pallas_call(
        _attn_kernel,
        grid=(_NCHUNK,),
        in_specs=[pl.BlockSpec((_BATCH, _EMB_DIM), lambda j: (0, 0))]
        + sub_specs + sub_specs,
        out_specs=pl.BlockSpec((_BATCH, _EMB_DIM), lambda j: (0, 0)),
        out_shape=jax.ShapeDtypeStruct((_BATCH, _EMB_DIM), jnp.float32),
        scratch_shapes=[
            pltpu.VMEM((_BATCH, _EMB_DIM), jnp.float32),
            pltpu.VMEM((_BATCH, _EMB_DIM), jnp.float32),
        ],
        compiler_params=pltpu.CompilerParams(
            dimension_semantics=("arbitrary",),
        ),
    )(encoded_state, *([keys] * _NSPLIT), *([values] * _NSPLIT))


# speedup vs baseline: 1.0250x; 1.0250x over previous
import jax
import jax.numpy as jnp
import numpy as np
from jax.experimental import pallas as pl
from jax.experimental.pallas import tpu as pltpu

_LTM_SIZE = 1000000
_EMB_DIM = 128
_BATCH = 32
_TEMPERATURE = 0.11 - float(np.log10(float(_LTM_SIZE))) * 0.01
_INV_T = 1.0 / _TEMPERATURE

_CHUNK = 10000
_NCHUNK = _LTM_SIZE // _CHUNK
_NSPLIT = 5
_SUB = _CHUNK // _NSPLIT


def _attn_kernel(q_ref, *refs):
    kv_refs = refs[:2 * _NSPLIT]
    o_ref, acc_ref, den_ref = refs[2 * _NSPLIT:]
    j = pl.program_id(0)

    q = q_ref[...]
    norm = jnp.sqrt(jnp.sum(q * q, axis=1, keepdims=True))
    qs = (q / jnp.maximum(norm, 1e-12)) * _INV_T

    pv = jnp.zeros((_BATCH, _EMB_DIM), jnp.float32)
    ps = jnp.zeros((_BATCH, 1), jnp.float32)
    for i in range(_NSPLIT):
        k_ref = kv_refs[i]
        v_ref = kv_refs[_NSPLIT + i]
        s = jax.lax.dot_general(
            qs, k_ref[...], (((1,), (1,)), ((), ())),
            preferred_element_type=jnp.float32,
        )
        p = jnp.exp(s - _INV_T)
        pv = pv + jax.lax.dot_general(
            p, v_ref[...], (((1,), (0,)), ((), ())),
            preferred_element_type=jnp.float32,
        )
        ps = ps + jnp.sum(p, axis=1, keepdims=True)
    psum = jnp.broadcast_to(ps, (_BATCH, _EMB_DIM))

    @pl.when(j == 0)
    def _init():
        acc_ref[...] = pv
        den_ref[...] = psum

    @pl.when(j != 0)
    def _accum():
        acc_ref[...] += pv
        den_ref[...] += psum

    @pl.when(j == _NCHUNK - 1)
    def _finish():
        o_ref[...] = acc_ref[...] / den_ref[...]


def _sub_spec(i):
    return pl.BlockSpec((_SUB, _EMB_DIM), lambda j, i=i: (_NSPLIT * j + i, 0))


def kernel(encoded_state, keys, values):
    sub_specs = [_sub_spec(i) for i in range(_NSPLIT)]
    return pl.pallas_call(
        _attn_kernel,
        grid=(_NCHUNK,),
        in_specs=[pl.BlockSpec((_BATCH, _EMB_DIM), lambda j: (0, 0))]
        + sub_specs + sub_specs,
        out_specs=pl.BlockSpec((_BATCH, _EMB_DIM), lambda j: (0, 0)),
        out_shape=jax.ShapeDtypeStruct((_BATCH, _EMB_DIM), jnp.float32),
        scratch_shapes=[
            pltpu.VMEM((_BATCH, _EMB_DIM), jnp.float32),
            pltpu.VMEM((_BATCH, _EMB_DIM), jnp.float32),
        ],
        compiler_params=pltpu.CompilerParams(
            dimension_semantics=("arbitrary",),
        ),
    )(encoded_state, *([keys] * _NSPLIT), *([values] * _NSPLIT))


# speedup vs baseline: 1.0323x; 1.0071x over previous
import jax
import jax.numpy as jnp
import numpy as np
from jax.experimental import pallas as pl
from jax.experimental.pallas import tpu as pltpu

_LTM_SIZE = 1000000
_EMB_DIM = 128
_BATCH = 32
_TEMPERATURE = 0.11 - float(np.log10(float(_LTM_SIZE))) * 0.01
_INV_T = 1.0 / _TEMPERATURE

_CHUNK = 10000
_NCHUNK = _LTM_SIZE // _CHUNK
_NSPLIT = 2
_SUB = _CHUNK // _NSPLIT


def _attn_kernel(q_ref, *refs):
    kv_refs = refs[:2 * _NSPLIT]
    o_ref, acc_ref, den_ref = refs[2 * _NSPLIT:]
    j = pl.program_id(0)

    q = q_ref[...]
    norm = jnp.sqrt(jnp.sum(q * q, axis=1, keepdims=True))
    qs = (q / jnp.maximum(norm, 1e-12)) * _INV_T

    pv = jnp.zeros((_BATCH, _EMB_DIM), jnp.float32)
    ps = jnp.zeros((_BATCH, 1), jnp.float32)
    for i in range(_NSPLIT):
        k_ref = kv_refs[i]
        v_ref = kv_refs[_NSPLIT + i]
        s = jax.lax.dot_general(
            qs, k_ref[...], (((1,), (1,)), ((), ())),
            preferred_element_type=jnp.float32,
        )
        p = jnp.exp(s - _INV_T)
        pv = pv + jax.lax.dot_general(
            p, v_ref[...], (((1,), (0,)), ((), ())),
            preferred_element_type=jnp.float32,
        )
        ps = ps + jnp.sum(p, axis=1, keepdims=True)
    psum = jnp.broadcast_to(ps, (_BATCH, _EMB_DIM))

    @pl.when(j == 0)
    def _init():
        acc_ref[...] = pv
        den_ref[...] = psum

    @pl.when(j != 0)
    def _accum():
        acc_ref[...] += pv
        den_ref[...] += psum

    @pl.when(j == _NCHUNK - 1)
    def _finish():
        o_ref[...] = acc_ref[...] / den_ref[...]


def _sub_spec(i):
    return pl.BlockSpec((_SUB, _EMB_DIM), lambda j, i=i: (_NSPLIT * j + i, 0))


def kernel(encoded_state, keys, values):
    sub_specs = [_sub_spec(i) for i in range(_NSPLIT)]
    return pl.pallas_call(
        _attn_kernel,
        grid=(_NCHUNK,),
        in_specs=[pl.BlockSpec((_BATCH, _EMB_DIM), lambda j: (0, 0))]
        + sub_specs + sub_specs,
        out_specs=pl.BlockSpec((_BATCH, _EMB_DIM), lambda j: (0, 0)),
        out_shape=jax.ShapeDtypeStruct((_BATCH, _EMB_DIM), jnp.float32),
        scratch_shapes=[
            pltpu.VMEM((_BATCH, _EMB_DIM), jnp.float32),
            pltpu.VMEM((_BATCH, _EMB_DIM), jnp.float32),
        ],
        compiler_params=pltpu.CompilerParams(
            dimension_semantics=("arbitrary",),
        ),
    )(encoded_state, *([keys] * _NSPLIT), *([values] * _NSPLIT))
